# baseline (speedup 1.0000x reference)
import functools

import jax
import jax.numpy as jnp
from jax import lax
from jax.experimental import pallas as pl
from jax.experimental.pallas import tpu as pltpu
from jax.experimental.pallas import tpu_sc as plsc

N = 10000
E = 320000
F = 128
NC = 2
NS = 16
EPT = E // NS
CHUNK = 128
NCH = 157
PAD = NCH * CHUNK - EPT
TAIL_REAL = EPT - (NCH - 1) * CHUNK
NTRI = (NCH - 1) // 3
EPTP = NCH * CHUNK
RB = 1000
GRID = N // RB
WT = 10
NA = N + 8
DEG_G = 10
DEG_NG = 15

_MESH = plsc.VectorSubcoreMesh(core_axis_name="c", subcore_axis_name="s")


def _fill_f32(ref, rows, cols, value):
    v = jnp.full((16,), value, jnp.float32)

    def body(i, _):
        for j in range(cols // 16):
            ref[i, pl.ds(j * 16, 16)] = v
        return 0

    lax.fori_loop(0, rows, body, 0, unroll=False)


def _fill_f32_1d(ref, start, n, value):
    v = jnp.full((16,), value, jnp.float32)

    def body(i, _):
        ref[pl.ds(start + i * 16, 16)] = v
        return 0

    lax.fori_loop(0, n // 16, body, 0, unroll=False)


def _deg_body(idx_hbm, deg_hbm, idx2d, ones_v, onest_v, zero_v, acc, sem):
    c = lax.axis_index("c")
    s = lax.axis_index("s")

    _fill_f32_1d(ones_v, 0, CHUNK, 1.0)
    _fill_f32_1d(onest_v, 0, TAIL_REAL, 1.0)
    _fill_f32_1d(onest_v, TAIL_REAL, CHUNK - TAIL_REAL, 0.0)
    _fill_f32_1d(zero_v, 0, 1024, 0.0)

    @pl.when(s < 10)
    def _():
        pltpu.sync_copy(zero_v.at[pl.ds(0, 1000)], acc.at[pl.ds(s * 1000, 1000)])

    @pl.when(s == 10)
    def _():
        pltpu.sync_copy(zero_v.at[pl.ds(0, 8)], acc.at[pl.ds(N, 8)])

    w = c * NS + s
    pltpu.sync_copy(idx_hbm.at[pl.ds(w * EPTP, EPTP)], idx2d)
    plsc.subcore_barrier()

    def group(g, _):
        for t in range(DEG_G):
            pltpu.async_copy(ones_v, acc.at[idx2d.at[pl.ds((g * DEG_G + t) * CHUNK, CHUNK)]], sem,
                             add=True)
        for t in range(DEG_G):
            pltpu.make_async_copy(ones_v, acc.at[idx2d.at[pl.ds((g * DEG_G + t) * CHUNK, CHUNK)]],
                                  sem).wait()
        return 0

    lax.fori_loop(0, DEG_NG, group, 0, unroll=False)
    for t in range(DEG_NG * DEG_G, NCH - 1):
        pltpu.async_copy(ones_v, acc.at[idx2d.at[pl.ds(t * CHUNK, CHUNK)]], sem, add=True)
    pltpu.async_copy(onest_v, acc.at[idx2d.at[pl.ds((NCH - 1) * CHUNK, CHUNK)]], sem, add=True)
    for t in range(DEG_NG * DEG_G, NCH - 1):
        pltpu.make_async_copy(ones_v, acc.at[idx2d.at[pl.ds(t * CHUNK, CHUNK)]], sem).wait()
    pltpu.make_async_copy(onest_v, acc.at[idx2d.at[pl.ds((NCH - 1) * CHUNK, CHUNK)]], sem).wait()

    plsc.subcore_barrier()

    @pl.when(s < 10)
    def _():
        pltpu.sync_copy(acc.at[pl.ds(s * 1000, 1000)], zero_v.at[pl.ds(0, 1000)])
        pltpu.sync_copy(zero_v.at[pl.ds(0, 1000)],
                        deg_hbm.at[pl.ds(c * N + s * 1000, 1000)])


@functools.partial(
    pl.kernel,
    out_type=jax.ShapeDtypeStruct((NC * N,), jnp.float32),
    mesh=_MESH,
    scratch_types=[
        pltpu.VMEM((EPTP,), jnp.int32),
        pltpu.VMEM((CHUNK,), jnp.float32),
        pltpu.VMEM((CHUNK,), jnp.float32),
        pltpu.VMEM((1024,), jnp.float32),
        pltpu.VMEM_SHARED((NA,), jnp.float32),
        pltpu.SemaphoreType.DMA,
    ],
)
def _degree_kernel(idx_hbm, deg_hbm, idx2d, ones_v, onest_v, zero_v, acc, sem):
    _deg_body(idx_hbm, deg_hbm, idx2d, ones_v, onest_v, zero_v, acc, sem)


def _agg_body(gidx_hbm, didx_hbm, h_hbm, out_hbm,
              sidx0, didx0, sidx1, didx1, rows_a, rows_b, rows_c, acc,
              gs_a, gs_b, gs_c, ss_a, ss_b, ss_c, isem0, isem1):
    c = lax.axis_index("c")
    s = lax.axis_index("s")
    w = c * NS + s

    def gath_start(ix, pos, rows, sem):
        pltpu.async_copy(h_hbm.at[ix.at[pl.ds(pos * CHUNK, CHUNK)]], rows, sem)

    def gath_wait(ix, pos, rows, sem):
        pltpu.make_async_copy(
            h_hbm.at[ix.at[pl.ds(pos * CHUNK, CHUNK)]], rows, sem).wait()

    def scat_start(ix, pos, rows, sem):
        pltpu.async_copy(rows, acc.at[ix.at[pl.ds(pos * CHUNK, CHUNK)]], sem,
                         add=True)

    def scat_wait(ix, pos, rows, sem):
        pltpu.make_async_copy(
            rows, acc.at[ix.at[pl.ds(pos * CHUNK, CHUNK)]], sem).wait()

    _fill_f32(rows_a, CHUNK, F, 0.0)

    @pl.when(s < WT)
    def _():
        for q in range(7):
            pltpu.async_copy(rows_a, acc.at[pl.ds(s * 1000 + q * 128, 128)],
                             ss_a)
        pltpu.async_copy(rows_a.at[pl.ds(0, 104), :],
                         acc.at[pl.ds(s * 1000 + 896, 104)], ss_a)

    @pl.when(s == WT)
    def _():
        pltpu.async_copy(rows_a.at[pl.ds(0, 8), :], acc.at[pl.ds(N, 8)], ss_a)

    pltpu.sync_copy(gidx_hbm.at[pl.ds(w * EPTP, 3 * CHUNK)], sidx0)
    pltpu.sync_copy(didx_hbm.at[pl.ds(s * EPTP, 3 * CHUNK)], didx0)
    pltpu.async_copy(gidx_hbm.at[pl.ds(w * EPTP + 3 * CHUNK, 3 * CHUNK)],
                     sidx1, isem1)
    pltpu.async_copy(didx_hbm.at[pl.ds(s * EPTP + 3 * CHUNK, 3 * CHUNK)],
                     didx1, isem1)

    @pl.when(s < WT)
    def _():
        for q in range(7):
            pltpu.make_async_copy(
                rows_a, acc.at[pl.ds(s * 1000 + q * 128, 128)], ss_a).wait()
        pltpu.make_async_copy(
            rows_a.at[pl.ds(0, 104), :],
            acc.at[pl.ds(s * 1000 + 896, 104)], ss_a).wait()

    @pl.when(s == WT)
    def _():
        pltpu.make_async_copy(rows_a.at[pl.ds(0, 8), :],
                              acc.at[pl.ds(N, 8)], ss_a).wait()

    plsc.subcore_barrier()

    gath_start(sidx0, 0, rows_a, gs_a)
    gath_start(sidx0, 1, rows_b, gs_b)

    def do_triple(j0, cur_s, cur_d, nxt_s, nxt_d, cur_isem, nxt_isem, prv_d):
        gath_wait(cur_s, 0, rows_a, gs_a)
        scat_start(cur_d, 0, rows_a, ss_a)
        nb = (j0 + 3) * CHUNK
        pltpu.make_async_copy(gidx_hbm.at[pl.ds(w * EPTP + nb, 3 * CHUNK)],
                              nxt_s, nxt_isem).wait()
        pltpu.make_async_copy(didx_hbm.at[pl.ds(s * EPTP + nb, 3 * CHUNK)],
                              nxt_d, nxt_isem).wait()

        @pl.when(j0 > 0)
        def _():
            scat_wait(prv_d, 2, rows_c, ss_c)

        gath_start(cur_s, 2, rows_c, gs_c)
        gath_wait(cur_s, 1, rows_b, gs_b)
        scat_start(cur_d, 1, rows_b, ss_b)
        scat_wait(cur_d, 0, rows_a, ss_a)
        gath_start(nxt_s, 0, rows_a, gs_a)
        gath_wait(cur_s, 2, rows_c, gs_c)
        scat_start(cur_d, 2, rows_c, ss_c)
        scat_wait(cur_d, 1, rows_b, ss_b)

        @pl.when(j0 + 4 <= NCH - 1)
        def _():
            gath_start(nxt_s, 1, rows_b, gs_b)

        @pl.when(j0 <= 3 * NTRI - 6)
        def _():
            pb = (j0 + 6) * CHUNK
            pltpu.async_copy(gidx_hbm.at[pl.ds(w * EPTP + pb, 3 * CHUNK)],
                             cur_s, cur_isem)
            pltpu.async_copy(didx_hbm.at[pl.ds(s * EPTP + pb, 3 * CHUNK)],
                             cur_d, cur_isem)

    def dtri(kk, _):
        do_triple(6 * kk, sidx0, didx0, sidx1, didx1, isem0, isem1, didx1)
        do_triple(6 * kk + 3, sidx1, didx1, sidx0, didx0, isem1, isem0, didx0)
        return 0

    lax.fori_loop(0, NTRI // 2, dtri, 0, unroll=False)

    gath_wait(sidx0, 0, rows_a, gs_a)
    scat_start(didx0, 0, rows_a, ss_a)
    scat_wait(didx1, 2, rows_c, ss_c)
    scat_wait(didx0, 0, rows_a, ss_a)

    plsc.subcore_barrier()

    @pl.when(s < WT)
    def _():
        for q in range(8):
            nr = 104 if q == 7 else 128
            buf = rows_a if q % 2 == 0 else rows_b
            sem = gs_a if q % 2 == 0 else gs_b
            r0 = s * 1000 + q * 128
            if q >= 2:
                pltpu.make_async_copy(
                    buf,
                    out_hbm.at[pl.ds(c * N + s * 1000 + (q - 2) * 128, 128), :],
                    sem).wait()
            pltpu.sync_copy(acc.at[pl.ds(r0, nr)], buf.at[pl.ds(0, nr), :])
            pltpu.async_copy(buf.at[pl.ds(0, nr), :],
                             out_hbm.at[pl.ds(c * N + r0, nr), :], sem)
        pltpu.make_async_copy(
            rows_a, out_hbm.at[pl.ds(c * N + s * 1000 + 6 * 128, 128), :],
            gs_a).wait()
        pltpu.make_async_copy(
            rows_b.at[pl.ds(0, 104), :],
            out_hbm.at[pl.ds(c * N + s * 1000 + 896, 104), :], gs_b).wait()


@functools.partial(
    pl.kernel,
    out_type=jax.ShapeDtypeStruct((NC * N, F), jnp.float32),
    mesh=_MESH,
    scratch_types=[
        pltpu.VMEM((3 * CHUNK,), jnp.int32),
        pltpu.VMEM((3 * CHUNK,), jnp.int32),
        pltpu.VMEM((3 * CHUNK,), jnp.int32),
        pltpu.VMEM((3 * CHUNK,), jnp.int32),
        pltpu.VMEM((CHUNK, F), jnp.float32),
        pltpu.VMEM((CHUNK, F), jnp.float32),
        pltpu.VMEM((CHUNK, F), jnp.float32),
        pltpu.VMEM_SHARED((NA, F), jnp.float32),
        pltpu.SemaphoreType.DMA,
        pltpu.SemaphoreType.DMA,
        pltpu.SemaphoreType.DMA,
        pltpu.SemaphoreType.DMA,
        pltpu.SemaphoreType.DMA,
        pltpu.SemaphoreType.DMA,
        pltpu.SemaphoreType.DMA,
        pltpu.SemaphoreType.DMA,
    ],
)
def _agg_kernel(gidx_hbm, didx_hbm, h_hbm, out_hbm,
                sidx0, didx0, sidx1, didx1, rows_a, rows_b, rows_c, acc,
                gs_a, gs_b, gs_c, ss_a, ss_b, ss_c, isem0, isem1):
    _agg_body(gidx_hbm, didx_hbm, h_hbm, out_hbm,
              sidx0, didx0, sidx1, didx1, rows_a, rows_b, rows_c, acc,
              gs_a, gs_b, gs_c, ss_a, ss_b, ss_c, isem0, isem1)


def _mm1_body(x_ref, deg_ref, w_ref, o_ref):
    inv = lax.rsqrt(jnp.maximum(deg_ref[...], 1.0))
    k = pl.program_id(0)
    o_ref[...] = jnp.dot(x_ref[...], w_ref[0],
                         preferred_element_type=jnp.float32) * inv


def _mm1(x, deg_out, w1):
    return pl.pallas_call(
        _mm1_body,
        grid=(NC, GRID),
        in_specs=[
            pl.BlockSpec((RB, F), lambda k, j: (j, 0)),
            pl.BlockSpec((RB, 1), lambda k, j: (j, 0)),
            pl.BlockSpec((1, F, F), lambda k, j: (k, 0, 0)),
        ],
        out_specs=pl.BlockSpec((RB, F), lambda k, j: (k * GRID + j, 0)),
        out_shape=jax.ShapeDtypeStruct((NC * N, F), jnp.float32),
    )(x, deg_out, w1)


def _mid_body(a_ref, din_ref, dout_ref, b_ref, w_ref, o_ref):
    inv_in = lax.rsqrt(jnp.maximum(din_ref[...], 1.0))
    inv_out = lax.rsqrt(jnp.maximum(dout_ref[...], 1.0))
    t = jnp.maximum(a_ref[...] * inv_in + b_ref[0], 0.0)
    o_ref[...] = jnp.dot(t, w_ref[0], preferred_element_type=jnp.float32) * inv_out


def _mid(agg1, deg_in, deg_out, b1, w2):
    return pl.pallas_call(
        _mid_body,
        grid=(NC, GRID),
        in_specs=[
            pl.BlockSpec((RB, F), lambda k, j: (k * GRID + j, 0)),
            pl.BlockSpec((RB, 1), lambda k, j: (j, 0)),
            pl.BlockSpec((RB, 1), lambda k, j: (j, 0)),
            pl.BlockSpec((1, 1, F), lambda k, j: (k, 0, 0)),
            pl.BlockSpec((1, F, F), lambda k, j: (k, 0, 0)),
        ],
        out_specs=pl.BlockSpec((RB, F), lambda k, j: (k * GRID + j, 0)),
        out_shape=jax.ShapeDtypeStruct((NC * N, F), jnp.float32),
    )(agg1, deg_in, deg_out, b1, w2)


def _head_body(ae_ref, ag_ref, din_ref, b_ref, wh_ref, bh_ref, y_ref, s_ref):
    inv_in = lax.rsqrt(jnp.maximum(din_ref[...], 1.0))
    hs = ae_ref[...] * inv_in + b_ref[0]
    s_ref[...] = jnp.dot(hs, wh_ref[0], preferred_element_type=jnp.float32) + bh_ref[0]
    z = ag_ref[...] * inv_in + b_ref[1]
    y_ref[...] = jnp.dot(z, wh_ref[1], preferred_element_type=jnp.float32) + bh_ref[1]


def _head(agg2, deg_in, b2, wh, bh):
    return pl.pallas_call(
        _head_body,
        grid=(GRID,),
        in_specs=[
            pl.BlockSpec((RB, F), lambda i: (i, 0)),
            pl.BlockSpec((RB, F), lambda i: (GRID + i, 0)),
            pl.BlockSpec((RB, 1), lambda i: (i, 0)),
            pl.BlockSpec((NC, 1, F), lambda i: (0, 0, 0)),
            pl.BlockSpec((NC, F, 1), lambda i: (0, 0, 0)),
            pl.BlockSpec((NC, 1, 1), lambda i: (0, 0, 0)),
        ],
        out_specs=[
            pl.BlockSpec((RB, 1), lambda i: (i, 0)),
            pl.BlockSpec((RB, 1), lambda i: (i, 0)),
        ],
        out_shape=[
            jax.ShapeDtypeStruct((N, 1), jnp.float32),
            jax.ShapeDtypeStruct((N, 1), jnp.float32),
        ],
    )(agg2, agg2, deg_in, b2, wh, bh)


def kernel(x, edge_index, W1e, b1e, W2e, b2e, Wfe, bfe, W1g, b1g, W2g, b2g, Wc, bc):
    src_t = edge_index[0].reshape(NS, EPT)
    dst_t = edge_index[1].reshape(NS, EPT)
    pad_src = jnp.broadcast_to((jnp.arange(PAD, dtype=jnp.int32) % 64)[None],
                               (NS, PAD))
    pad_dst = jnp.broadcast_to((N + jnp.arange(PAD, dtype=jnp.int32) % 8)[None],
                               (NS, PAD))
    src_p = jnp.concatenate([src_t, pad_src], 1).reshape(NS * EPTP)
    dst_p = jnp.concatenate([dst_t, pad_dst], 1).reshape(NS * EPTP)
    opad = jnp.zeros((2 * CHUNK,), jnp.int32)
    deg_idx = jnp.concatenate([src_p, dst_p])
    gat_idx = jnp.concatenate([src_p, src_p + N, opad])
    dst_p = jnp.concatenate([dst_p, opad])

    w1 = jnp.stack([W1e, W1g])
    b1 = jnp.stack([b1e, b1g])[:, None, :]
    w2 = jnp.stack([W2e, W2g])
    b2 = jnp.stack([b2e, b2g])[:, None, :]
    wh = jnp.stack([Wfe, Wc])
    bh = jnp.stack([bfe, bc])[:, None, :]

    deg = _degree_kernel(deg_idx)
    deg_out = deg[:N].reshape(N, 1)
    deg_in = deg[N:].reshape(N, 1)

    h1 = _mm1(x, deg_out, w1)
    agg1 = _agg_kernel(gat_idx, dst_p, h1)
    h2 = _mid(agg1, deg_in, deg_out, b1, w2)
    agg2 = _agg_kernel(gat_idx, dst_p, h2)
    y, s = _head(agg2, deg_in, b2, wh, bh)
    return (y, s)


# speedup vs baseline: 1.0305x; 1.0305x over previous
import functools

import jax
import jax.numpy as jnp
from jax import lax
from jax.experimental import pallas as pl
from jax.experimental.pallas import tpu as pltpu
from jax.experimental.pallas import tpu_sc as plsc

N = 10000
E = 320000
F = 128
NC = 2
NS = 16
EPT = E // NS
CHUNK = 128
NCH = 157
PAD = NCH * CHUNK - EPT
TAIL_REAL = EPT - (NCH - 1) * CHUNK
NTRI = (NCH - 1) // 3
EPTP = NCH * CHUNK
RB = 1000
GRID = N // RB
WT = 10
NA = N + 8
DEG_G = 10
DEG_NG = 15

_MESH = plsc.VectorSubcoreMesh(core_axis_name="c", subcore_axis_name="s")


def _fill_f32(ref, rows, cols, value):
    v = jnp.full((16,), value, jnp.float32)

    def body(i, _):
        for j in range(cols // 16):
            ref[i, pl.ds(j * 16, 16)] = v
        return 0

    lax.fori_loop(0, rows, body, 0, unroll=False)


def _fill_f32_1d(ref, start, n, value):
    v = jnp.full((16,), value, jnp.float32)

    def body(i, _):
        ref[pl.ds(start + i * 16, 16)] = v
        return 0

    lax.fori_loop(0, n // 16, body, 0, unroll=False)


def _deg_body(idx_hbm, deg_hbm, idx2d, ones_v, onest_v, zero_v, acc, sem):
    c = lax.axis_index("c")
    s = lax.axis_index("s")

    _fill_f32_1d(ones_v, 0, CHUNK, 1.0)
    _fill_f32_1d(onest_v, 0, TAIL_REAL, 1.0)
    _fill_f32_1d(onest_v, TAIL_REAL, CHUNK - TAIL_REAL, 0.0)
    _fill_f32_1d(zero_v, 0, 1024, 0.0)

    @pl.when(s < 10)
    def _():
        pltpu.sync_copy(zero_v.at[pl.ds(0, 1000)], acc.at[pl.ds(s * 1000, 1000)])

    @pl.when(s == 10)
    def _():
        pltpu.sync_copy(zero_v.at[pl.ds(0, 8)], acc.at[pl.ds(N, 8)])

    w = c * NS + s
    pltpu.sync_copy(idx_hbm.at[pl.ds(w * EPTP, EPTP)], idx2d)
    plsc.subcore_barrier()

    def group(g, _):
        for t in range(DEG_G):
            pltpu.async_copy(ones_v, acc.at[idx2d.at[pl.ds((g * DEG_G + t) * CHUNK, CHUNK)]], sem,
                             add=True)
        for t in range(DEG_G):
            pltpu.make_async_copy(ones_v, acc.at[idx2d.at[pl.ds((g * DEG_G + t) * CHUNK, CHUNK)]],
                                  sem).wait()
        return 0

    lax.fori_loop(0, DEG_NG, group, 0, unroll=False)
    for t in range(DEG_NG * DEG_G, NCH - 1):
        pltpu.async_copy(ones_v, acc.at[idx2d.at[pl.ds(t * CHUNK, CHUNK)]], sem, add=True)
    pltpu.async_copy(onest_v, acc.at[idx2d.at[pl.ds((NCH - 1) * CHUNK, CHUNK)]], sem, add=True)
    for t in range(DEG_NG * DEG_G, NCH - 1):
        pltpu.make_async_copy(ones_v, acc.at[idx2d.at[pl.ds(t * CHUNK, CHUNK)]], sem).wait()
    pltpu.make_async_copy(onest_v, acc.at[idx2d.at[pl.ds((NCH - 1) * CHUNK, CHUNK)]], sem).wait()

    plsc.subcore_barrier()

    @pl.when(s < 10)
    def _():
        pltpu.sync_copy(acc.at[pl.ds(s * 1000, 1000)], zero_v.at[pl.ds(0, 1000)])
        pltpu.sync_copy(zero_v.at[pl.ds(0, 1000)],
                        deg_hbm.at[pl.ds(c * N + s * 1000, 1000)])


@functools.partial(
    pl.kernel,
    out_type=jax.ShapeDtypeStruct((NC * N,), jnp.float32),
    mesh=_MESH,
    scratch_types=[
        pltpu.VMEM((EPTP,), jnp.int32),
        pltpu.VMEM((CHUNK,), jnp.float32),
        pltpu.VMEM((CHUNK,), jnp.float32),
        pltpu.VMEM((1024,), jnp.float32),
        pltpu.VMEM_SHARED((NA,), jnp.float32),
        pltpu.SemaphoreType.DMA,
    ],
)
def _degree_kernel(idx_hbm, deg_hbm, idx2d, ones_v, onest_v, zero_v, acc, sem):
    _deg_body(idx_hbm, deg_hbm, idx2d, ones_v, onest_v, zero_v, acc, sem)


def _agg_body(gidx_hbm, didx_hbm, h_hbm, out_hbm,
              sidx0, didx0, sidx1, didx1, rows_a, rows_b, rows_c, acc,
              gs_a, gs_b, gs_c, ss_a, ss_b, ss_c, isem0, isem1):
    c = lax.axis_index("c")
    s = lax.axis_index("s")
    w = c * NS + s

    def gath_start(ix, pos, rows, sem):
        pltpu.async_copy(h_hbm.at[ix.at[pl.ds(pos * CHUNK, CHUNK)]], rows, sem)

    def gath_wait(ix, pos, rows, sem):
        pltpu.make_async_copy(
            h_hbm.at[ix.at[pl.ds(pos * CHUNK, CHUNK)]], rows, sem).wait()

    def scat_start(ix, pos, rows, sem):
        pltpu.async_copy(rows, acc.at[ix.at[pl.ds(pos * CHUNK, CHUNK)]], sem,
                         add=True)

    def scat_wait(ix, pos, rows, sem):
        pltpu.make_async_copy(
            rows, acc.at[ix.at[pl.ds(pos * CHUNK, CHUNK)]], sem).wait()

    _fill_f32(rows_a, CHUNK, F, 0.0)

    @pl.when(s < WT)
    def _():
        for q in range(7):
            pltpu.async_copy(rows_a, acc.at[pl.ds(s * 1000 + q * 128, 128)],
                             ss_a)
        pltpu.async_copy(rows_a.at[pl.ds(0, 104), :],
                         acc.at[pl.ds(s * 1000 + 896, 104)], ss_a)

    @pl.when(s == WT)
    def _():
        pltpu.async_copy(rows_a.at[pl.ds(0, 8), :], acc.at[pl.ds(N, 8)], ss_a)

    pltpu.sync_copy(gidx_hbm.at[pl.ds(w * EPTP, 3 * CHUNK)], sidx0)
    pltpu.sync_copy(didx_hbm.at[pl.ds(s * EPTP, 3 * CHUNK)], didx0)
    pltpu.async_copy(gidx_hbm.at[pl.ds(w * EPTP + 3 * CHUNK, 3 * CHUNK)],
                     sidx1, isem1)
    pltpu.async_copy(didx_hbm.at[pl.ds(s * EPTP + 3 * CHUNK, 3 * CHUNK)],
                     didx1, isem1)

    @pl.when(s < WT)
    def _():
        for q in range(7):
            pltpu.make_async_copy(
                rows_a, acc.at[pl.ds(s * 1000 + q * 128, 128)], ss_a).wait()
        pltpu.make_async_copy(
            rows_a.at[pl.ds(0, 104), :],
            acc.at[pl.ds(s * 1000 + 896, 104)], ss_a).wait()

    @pl.when(s == WT)
    def _():
        pltpu.make_async_copy(rows_a.at[pl.ds(0, 8), :],
                              acc.at[pl.ds(N, 8)], ss_a).wait()

    plsc.subcore_barrier()

    gath_start(sidx0, 0, rows_a, gs_a)
    gath_start(sidx0, 1, rows_b, gs_b)

    def do_triple(j0, cur_s, cur_d, nxt_s, nxt_d, cur_isem, nxt_isem, prv_d):
        gath_wait(cur_s, 0, rows_a, gs_a)
        scat_start(cur_d, 0, rows_a, ss_a)
        nb = (j0 + 3) * CHUNK
        pltpu.make_async_copy(gidx_hbm.at[pl.ds(w * EPTP + nb, 3 * CHUNK)],
                              nxt_s, nxt_isem).wait()
        pltpu.make_async_copy(didx_hbm.at[pl.ds(s * EPTP + nb, 3 * CHUNK)],
                              nxt_d, nxt_isem).wait()

        @pl.when(j0 > 0)
        def _():
            scat_wait(prv_d, 2, rows_c, ss_c)

        gath_start(cur_s, 2, rows_c, gs_c)
        gath_wait(cur_s, 1, rows_b, gs_b)
        scat_start(cur_d, 1, rows_b, ss_b)
        scat_wait(cur_d, 0, rows_a, ss_a)
        gath_start(nxt_s, 0, rows_a, gs_a)
        gath_wait(cur_s, 2, rows_c, gs_c)
        scat_start(cur_d, 2, rows_c, ss_c)
        scat_wait(cur_d, 1, rows_b, ss_b)

        @pl.when(j0 + 4 <= NCH - 1)
        def _():
            gath_start(nxt_s, 1, rows_b, gs_b)

        @pl.when(j0 <= 3 * NTRI - 6)
        def _():
            pb = (j0 + 6) * CHUNK
            pltpu.async_copy(gidx_hbm.at[pl.ds(w * EPTP + pb, 3 * CHUNK)],
                             cur_s, cur_isem)
            pltpu.async_copy(didx_hbm.at[pl.ds(s * EPTP + pb, 3 * CHUNK)],
                             cur_d, cur_isem)

    def dtri(kk, _):
        do_triple(6 * kk, sidx0, didx0, sidx1, didx1, isem0, isem1, didx1)
        do_triple(6 * kk + 3, sidx1, didx1, sidx0, didx0, isem1, isem0, didx0)
        return 0

    lax.fori_loop(0, NTRI // 2, dtri, 0, unroll=False)

    gath_wait(sidx0, 0, rows_a, gs_a)
    scat_start(didx0, 0, rows_a, ss_a)
    scat_wait(didx1, 2, rows_c, ss_c)
    scat_wait(didx0, 0, rows_a, ss_a)

    plsc.subcore_barrier()

    @pl.when(s < WT)
    def _():
        for q in range(8):
            nr = 104 if q == 7 else 128
            buf = rows_a if q % 2 == 0 else rows_b
            sem = gs_a if q % 2 == 0 else gs_b
            r0 = s * 1000 + q * 128
            if q >= 2:
                pltpu.make_async_copy(
                    buf, out_hbm.at[c, pl.ds(s * 1000 + (q - 2) * 128, 128), :],
                    sem).wait()
            pltpu.sync_copy(acc.at[pl.ds(r0, nr)], buf.at[pl.ds(0, nr), :])
            pltpu.async_copy(buf.at[pl.ds(0, nr), :],
                             out_hbm.at[c, pl.ds(r0, nr), :], sem)
        pltpu.make_async_copy(rows_a,
                              out_hbm.at[c, pl.ds(s * 1000 + 6 * 128, 128), :],
                              gs_a).wait()
        pltpu.make_async_copy(rows_b.at[pl.ds(0, 104), :],
                              out_hbm.at[c, pl.ds(s * 1000 + 896, 104), :],
                              gs_b).wait()


@functools.partial(
    pl.kernel,
    out_type=jax.ShapeDtypeStruct((NC, N, F), jnp.float32),
    mesh=_MESH,
    scratch_types=[
        pltpu.VMEM((3 * CHUNK,), jnp.int32),
        pltpu.VMEM((3 * CHUNK,), jnp.int32),
        pltpu.VMEM((3 * CHUNK,), jnp.int32),
        pltpu.VMEM((3 * CHUNK,), jnp.int32),
        pltpu.VMEM((CHUNK, F), jnp.float32),
        pltpu.VMEM((CHUNK, F), jnp.float32),
        pltpu.VMEM((CHUNK, F), jnp.float32),
        pltpu.VMEM_SHARED((NA, F), jnp.float32),
        pltpu.SemaphoreType.DMA,
        pltpu.SemaphoreType.DMA,
        pltpu.SemaphoreType.DMA,
        pltpu.SemaphoreType.DMA,
        pltpu.SemaphoreType.DMA,
        pltpu.SemaphoreType.DMA,
        pltpu.SemaphoreType.DMA,
        pltpu.SemaphoreType.DMA,
    ],
)
def _agg_kernel(gidx_hbm, didx_hbm, h_hbm, out_hbm,
                sidx0, didx0, sidx1, didx1, rows_a, rows_b, rows_c, acc,
                gs_a, gs_b, gs_c, ss_a, ss_b, ss_c, isem0, isem1):
    _agg_body(gidx_hbm, didx_hbm, h_hbm, out_hbm,
              sidx0, didx0, sidx1, didx1, rows_a, rows_b, rows_c, acc,
              gs_a, gs_b, gs_c, ss_a, ss_b, ss_c, isem0, isem1)


def _mm1_body(x_ref, deg_ref, w_ref, o_ref):
    inv = lax.rsqrt(jnp.maximum(deg_ref[...], 1.0))
    xs = x_ref[...]
    o_ref[0] = jnp.dot(xs, w_ref[0], preferred_element_type=jnp.float32) * inv
    o_ref[1] = jnp.dot(xs, w_ref[1], preferred_element_type=jnp.float32) * inv


def _mm1(x, deg_out, w1):
    return pl.pallas_call(
        _mm1_body,
        grid=(GRID,),
        in_specs=[
            pl.BlockSpec((RB, F), lambda i: (i, 0)),
            pl.BlockSpec((RB, 1), lambda i: (i, 0)),
            pl.BlockSpec((NC, F, F), lambda i: (0, 0, 0)),
        ],
        out_specs=pl.BlockSpec((NC, RB, F), lambda i: (0, i, 0)),
        out_shape=jax.ShapeDtypeStruct((NC, N, F), jnp.float32),
    )(x, deg_out, w1)


def _mid_body(a_ref, din_ref, dout_ref, b_ref, w_ref, o_ref):
    inv_in = lax.rsqrt(jnp.maximum(din_ref[...], 1.0))
    inv_out = lax.rsqrt(jnp.maximum(dout_ref[...], 1.0))
    for k in range(NC):
        t = jnp.maximum(a_ref[k] * inv_in + b_ref[k], 0.0)
        o_ref[k] = jnp.dot(t, w_ref[k], preferred_element_type=jnp.float32) * inv_out


def _mid(agg1, deg_in, deg_out, b1, w2):
    return pl.pallas_call(
        _mid_body,
        grid=(GRID,),
        in_specs=[
            pl.BlockSpec((NC, RB, F), lambda i: (0, i, 0)),
            pl.BlockSpec((RB, 1), lambda i: (i, 0)),
            pl.BlockSpec((RB, 1), lambda i: (i, 0)),
            pl.BlockSpec((NC, 1, F), lambda i: (0, 0, 0)),
            pl.BlockSpec((NC, F, F), lambda i: (0, 0, 0)),
        ],
        out_specs=pl.BlockSpec((NC, RB, F), lambda i: (0, i, 0)),
        out_shape=jax.ShapeDtypeStruct((NC, N, F), jnp.float32),
    )(agg1, deg_in, deg_out, b1, w2)


def _head_body(a_ref, din_ref, b_ref, wh_ref, bh_ref, y_ref, s_ref):
    inv_in = lax.rsqrt(jnp.maximum(din_ref[...], 1.0))
    hs = a_ref[0] * inv_in + b_ref[0]
    s_ref[...] = jnp.dot(hs, wh_ref[0], preferred_element_type=jnp.float32) + bh_ref[0]
    z = a_ref[1] * inv_in + b_ref[1]
    y_ref[...] = jnp.dot(z, wh_ref[1], preferred_element_type=jnp.float32) + bh_ref[1]


def _head(agg2, deg_in, b2, wh, bh):
    return pl.pallas_call(
        _head_body,
        grid=(GRID,),
        in_specs=[
            pl.BlockSpec((NC, RB, F), lambda i: (0, i, 0)),
            pl.BlockSpec((RB, 1), lambda i: (i, 0)),
            pl.BlockSpec((NC, 1, F), lambda i: (0, 0, 0)),
            pl.BlockSpec((NC, F, 1), lambda i: (0, 0, 0)),
            pl.BlockSpec((NC, 1, 1), lambda i: (0, 0, 0)),
        ],
        out_specs=[
            pl.BlockSpec((RB, 1), lambda i: (i, 0)),
            pl.BlockSpec((RB, 1), lambda i: (i, 0)),
        ],
        out_shape=[
            jax.ShapeDtypeStruct((N, 1), jnp.float32),
            jax.ShapeDtypeStruct((N, 1), jnp.float32),
        ],
    )(agg2, deg_in, b2, wh, bh)


def kernel(x, edge_index, W1e, b1e, W2e, b2e, Wfe, bfe, W1g, b1g, W2g, b2g, Wc, bc):
    src_t = edge_index[0].reshape(NS, EPT)
    dst_t = edge_index[1].reshape(NS, EPT)
    pad_src = jnp.broadcast_to((jnp.arange(PAD, dtype=jnp.int32) % 64)[None],
                               (NS, PAD))
    pad_dst = jnp.broadcast_to((N + jnp.arange(PAD, dtype=jnp.int32) % 8)[None],
                               (NS, PAD))
    src_p = jnp.concatenate([src_t, pad_src], 1).reshape(NS * EPTP)
    dst_p = jnp.concatenate([dst_t, pad_dst], 1).reshape(NS * EPTP)
    opad = jnp.zeros((2 * CHUNK,), jnp.int32)
    deg_idx = jnp.concatenate([src_p, dst_p])
    gat_idx = jnp.concatenate([src_p, src_p + N, opad])
    dst_p = jnp.concatenate([dst_p, opad])

    w1 = jnp.stack([W1e, W1g])
    b1 = jnp.stack([b1e, b1g])[:, None, :]
    w2 = jnp.stack([W2e, W2g])
    b2 = jnp.stack([b2e, b2g])[:, None, :]
    wh = jnp.stack([Wfe, Wc])
    bh = jnp.stack([bfe, bc])[:, None, :]

    deg = _degree_kernel(deg_idx)
    deg_out = deg[:N].reshape(N, 1)
    deg_in = deg[N:].reshape(N, 1)

    h1 = _mm1(x, deg_out, w1)
    agg1 = _agg_kernel(gat_idx, dst_p, h1.reshape(NC * N, F))
    h2 = _mid(agg1, deg_in, deg_out, b1, w2)
    agg2 = _agg_kernel(gat_idx, dst_p, h2.reshape(NC * N, F))
    y, s = _head(agg2, deg_in, b2, wh, bh)
    return (y, s)
